# baseline (speedup 1.0000x reference)
import jax
import jax.numpy as jnp
from jax import lax
from jax.experimental import pallas as pl
from jax.experimental.pallas import tpu as pltpu

_K = 9
_DIL = 2
_KSEL = (_K - 1) * _DIL + 1


def _knn_sel_kernel(q_ref, k_ref, khalf_ref, edge_ref):
    q = q_ref[0]
    kn = k_ref[0]
    tq = q.shape[1]
    nk = kn.shape[1]

    gram = lax.dot_general(q, kn, (((0,), (0,)), ((), ())),
                           preferred_element_type=jnp.float32)
    d_full = khalf_ref[0] - gram

    lane = lax.broadcasted_iota(jnp.int32, (1, 128), 1).astype(jnp.float32)
    big = jnp.float32(2 ** 30)
    inf = jnp.float32(jnp.inf)
    out_iota = lax.broadcasted_iota(jnp.int32, (1, _K), 1)
    acc = jnp.zeros((tq, _K), jnp.float32)
    nt = nk // 128

    tiles = [d_full[:, j * 128:(j + 1) * 128] for j in range(nt)]
    tile_gi = [lane + jnp.float32(j * 128) for j in range(nt)]

    sel = None
    for r in range(_KSEL):
        if sel is not None:
            tiles = [jnp.where(tile_gi[j] == sel, inf, tiles[j])
                     for j in range(nt)]
        level = [(jnp.minimum(tiles[j], tiles[j + 1]),
                  jnp.where(tiles[j] <= tiles[j + 1],
                            jnp.float32(j), jnp.float32(j + 1)))
                 for j in range(0, nt, 2)]
        while len(level) > 1:
            level = [(jnp.minimum(level[a][0], level[a + 1][0]),
                      jnp.where(level[a][0] <= level[a + 1][0],
                                level[a][1], level[a + 1][1]))
                     for a in range(0, len(level), 2)]
        v0, t0 = level[0]

        vmin = jnp.min(v0, axis=-1, keepdims=True)
        gi = t0 * 128 + lane
        cand = jnp.where(v0 <= vmin, gi, big)
        sel = jnp.min(cand, axis=-1, keepdims=True)
        if r % _DIL == 0:
            acc = jnp.where(out_iota == (r // _DIL), sel, acc)

    edge_ref[0, 0] = acc.astype(jnp.int32)
    center = pl.program_id(1) * tq + lax.broadcasted_iota(jnp.int32, (tq, _K), 0)
    edge_ref[1, 0] = center


def _l2_normalize(x_bcn, eps=1e-12):
    ssq = jnp.sum(x_bcn * x_bcn, axis=1, keepdims=True)
    return x_bcn * lax.rsqrt(jnp.maximum(ssq, eps * eps))


def kernel(x):
    B, C, N, W = x.shape
    assert W == 1
    xn = _l2_normalize(x[..., 0].astype(jnp.float32))
    khalf = 0.5 * jnp.sum(xn * xn, axis=1, keepdims=True)

    tq = 256 if N % 256 == 0 else 128
    assert N % tq == 0 and _KSEL <= N

    return pl.pallas_call(
        _knn_sel_kernel,
        out_shape=jax.ShapeDtypeStruct((2, B, N, _K), jnp.int32),
        grid=(B, N // tq),
        in_specs=[
            pl.BlockSpec((1, C, tq), lambda b, t: (b, 0, t)),
            pl.BlockSpec((1, C, N), lambda b, t: (b, 0, 0)),
            pl.BlockSpec((1, 1, N), lambda b, t: (b, 0, 0)),
        ],
        out_specs=pl.BlockSpec((2, 1, tq, _K), lambda b, t: (0, b, t, 0)),
        compiler_params=pltpu.CompilerParams(
            dimension_semantics=("parallel", "parallel"),
            vmem_limit_bytes=64 * 1024 * 1024,
        ),
    )(xn, xn, khalf)


# speedup vs baseline: 1.0711x; 1.0711x over previous
import jax
import jax.numpy as jnp
from jax import lax
from jax.experimental import pallas as pl
from jax.experimental.pallas import tpu as pltpu

_K = 9
_DIL = 2
_KSEL = (_K - 1) * _DIL + 1


def _knn_sel_kernel(q_ref, k_ref, khalf_ref, edge_ref):
    q = q_ref[0]
    kn = k_ref[0]
    tq = q.shape[1]
    nk = kn.shape[1]

    gram = lax.dot_general(q, kn, (((0,), (0,)), ((), ())),
                           preferred_element_type=jnp.float32)
    d_full = khalf_ref[0] - gram

    lane = lax.broadcasted_iota(jnp.int32, (1, 128), 1).astype(jnp.float32)
    big = jnp.float32(2 ** 30)
    inf = jnp.float32(jnp.inf)
    out_iota = lax.broadcasted_iota(jnp.int32, (1, _K), 1)
    acc = jnp.zeros((tq, _K), jnp.float32)
    nt = nk // 128

    tiles = [d_full[:, j * 128:(j + 1) * 128] for j in range(nt)]
    tile_gi = [lane + jnp.float32(j * 128) for j in range(nt)]

    sel = None
    for r in range(_KSEL):
        if sel is not None:
            tiles = [jnp.where(tile_gi[j] == sel, inf, tiles[j])
                     for j in range(nt)]
        level = [(jnp.minimum(tiles[j], tiles[j + 1]),
                  jnp.where(tiles[j] <= tiles[j + 1],
                            jnp.float32(j), jnp.float32(j + 1)))
                 for j in range(0, nt, 2)]
        while len(level) > 1:
            level = [(jnp.minimum(level[a][0], level[a + 1][0]),
                      jnp.where(level[a][0] <= level[a + 1][0],
                                level[a][1], level[a + 1][1]))
                     for a in range(0, len(level), 2)]
        v0, t0 = level[0]

        vmin = jnp.min(v0, axis=-1, keepdims=True)
        gi = t0 * 128 + lane
        cand = jnp.where(v0 <= vmin, gi, big)
        sel = jnp.min(cand, axis=-1, keepdims=True)
        if r % _DIL == 0:
            acc = jnp.where(out_iota == (r // _DIL), sel, acc)

    edge_ref[0, 0] = acc.astype(jnp.int32)
    center = pl.program_id(1) * tq + lax.broadcasted_iota(jnp.int32, (tq, _K), 0)
    edge_ref[1, 0] = center


def _l2_normalize(x_bcn, eps=1e-12):
    ssq = jnp.sum(x_bcn * x_bcn, axis=1, keepdims=True)
    return x_bcn * lax.rsqrt(jnp.maximum(ssq, eps * eps))


def kernel(x):
    B, C, N, W = x.shape
    assert W == 1
    xn = _l2_normalize(x[..., 0].astype(jnp.float32))
    khalf = 0.5 * jnp.sum(xn * xn, axis=1, keepdims=True)

    tq = 512 if N % 512 == 0 else (256 if N % 256 == 0 else 128)
    assert N % tq == 0 and _KSEL <= N

    return pl.pallas_call(
        _knn_sel_kernel,
        out_shape=jax.ShapeDtypeStruct((2, B, N, _K), jnp.int32),
        grid=(B, N // tq),
        in_specs=[
            pl.BlockSpec((1, C, tq), lambda b, t: (b, 0, t)),
            pl.BlockSpec((1, C, N), lambda b, t: (b, 0, 0)),
            pl.BlockSpec((1, 1, N), lambda b, t: (b, 0, 0)),
        ],
        out_specs=pl.BlockSpec((2, 1, tq, _K), lambda b, t: (0, b, t, 0)),
        compiler_params=pltpu.CompilerParams(
            dimension_semantics=("parallel", "parallel"),
            vmem_limit_bytes=64 * 1024 * 1024,
        ),
    )(xn, xn, khalf)
